# baseline (speedup 1.0000x reference)
import jax
import jax.numpy as jnp
from jax.experimental import pallas as pl
from jax.experimental.pallas import tpu as pltpu

_HIGHEST = jax.lax.Precision.HIGHEST
_LANES = 128
_BLOCK_ROWS = 2048


def _wl_body(bd_ref, b_ref, x_ref, o_ref):
    y = jnp.dot(x_ref[...], bd_ref[...], preferred_element_type=jnp.float32)
    o_ref[...] = jnp.maximum(y + b_ref[0:1, :], 0.0).astype(o_ref.dtype)


def kernel(x, corr_adj, w, bias):
    B, T, C = x.shape
    dtype = x.dtype
    N = B * T * C

    m = jnp.matmul(corr_adj.astype(jnp.float32), w.astype(jnp.float32),
                   precision=_HIGHEST)

    win = T * C
    groups = _LANES // C
    bd = jnp.kron(jnp.eye(groups, dtype=jnp.float32), m)

    bias_f32 = (jnp.zeros((T, C), jnp.float32) if bias is None
                else bias.astype(jnp.float32))
    brow = jnp.tile(bias_f32.reshape(-1), _LANES // win)
    bblk = jnp.broadcast_to(brow, (8, _LANES))

    rows = N // _LANES
    x2 = x.reshape(rows, _LANES)
    blk = min(_BLOCK_ROWS, rows)
    grid = (pl.cdiv(rows, blk),)

    out2 = pl.pallas_call(
        _wl_body,
        out_shape=jax.ShapeDtypeStruct((rows, _LANES), dtype),
        grid=grid,
        in_specs=[
            pl.BlockSpec((_LANES, _LANES), lambda i: (0, 0)),
            pl.BlockSpec((8, _LANES), lambda i: (0, 0)),
            pl.BlockSpec((blk, _LANES), lambda i: (i, 0)),
        ],
        out_specs=pl.BlockSpec((blk, _LANES), lambda i: (i, 0)),
        compiler_params=pltpu.CompilerParams(
            dimension_semantics=("parallel",)),
    )(bd, bblk, x2)

    return out2.reshape(B, T, C)


# speedup vs baseline: 104.6542x; 104.6542x over previous
import jax
import jax.numpy as jnp
from jax.experimental import pallas as pl
from jax.experimental.pallas import tpu as pltpu

_HIGHEST = jax.lax.Precision.HIGHEST
_BLOCK_B = 4096


def _wl_body(mt_ref, b_ref, x_ref, o_ref):
    T = x_ref.shape[0]
    nb = x_ref.shape[2]
    mt = mt_ref[...]
    for t in range(T):
        y = jnp.dot(mt, x_ref[t], preferred_element_type=jnp.float32)
        bt = jnp.broadcast_to(b_ref[t][:, 0:1], y.shape)
        o_ref[t] = jnp.maximum(y + bt, 0.0).astype(o_ref.dtype)


def kernel(x, corr_adj, w, bias):
    B, T, C = x.shape
    dtype = x.dtype

    m = jnp.matmul(corr_adj.astype(jnp.float32), w.astype(jnp.float32),
                   precision=_HIGHEST)
    mt = m.T

    bias_f32 = (jnp.zeros((T, C), jnp.float32) if bias is None
                else bias.astype(jnp.float32))
    b_bc = jnp.broadcast_to(bias_f32[:, :, None], (T, C, 128))

    x_t = jnp.transpose(x, (1, 2, 0))
    blk = min(_BLOCK_B, B)
    grid = (pl.cdiv(B, blk),)

    out_t = pl.pallas_call(
        _wl_body,
        out_shape=jax.ShapeDtypeStruct((T, C, B), dtype),
        grid=grid,
        in_specs=[
            pl.BlockSpec((C, C), lambda i: (0, 0)),
            pl.BlockSpec((T, C, 128), lambda i: (0, 0, 0)),
            pl.BlockSpec((T, C, blk), lambda i: (0, 0, i)),
        ],
        out_specs=pl.BlockSpec((T, C, blk), lambda i: (0, 0, i)),
        compiler_params=pltpu.CompilerParams(
            dimension_semantics=("parallel",)),
    )(mt, b_bc, x_t)

    return jnp.transpose(out_t, (2, 0, 1))


# speedup vs baseline: 138.6942x; 1.3253x over previous
import jax
import jax.numpy as jnp
from jax.experimental import pallas as pl
from jax.experimental.pallas import tpu as pltpu

_HIGHEST = jax.lax.Precision.HIGHEST
_BLOCK_B = 8192


def _wl_body(mt_ref, b_ref, x_ref, o_ref):
    T = x_ref.shape[0]
    nb = x_ref.shape[2]
    mt = mt_ref[...]
    for t in range(T):
        y = jnp.dot(mt, x_ref[t], preferred_element_type=jnp.float32)
        bt = jnp.broadcast_to(b_ref[t][:, 0:1], y.shape)
        o_ref[t] = jnp.maximum(y + bt, 0.0).astype(o_ref.dtype)


def kernel(x, corr_adj, w, bias):
    B, T, C = x.shape
    dtype = x.dtype

    m = jnp.matmul(corr_adj.astype(jnp.float32), w.astype(jnp.float32),
                   precision=_HIGHEST)
    mt = m.T

    bias_f32 = (jnp.zeros((T, C), jnp.float32) if bias is None
                else bias.astype(jnp.float32))
    b_bc = jnp.broadcast_to(bias_f32[:, :, None], (T, C, 128))

    x_t = jnp.transpose(x, (1, 2, 0))
    blk = min(_BLOCK_B, B)
    grid = (pl.cdiv(B, blk),)

    out_t = pl.pallas_call(
        _wl_body,
        out_shape=jax.ShapeDtypeStruct((T, C, B), dtype),
        grid=grid,
        in_specs=[
            pl.BlockSpec((C, C), lambda i: (0, 0)),
            pl.BlockSpec((T, C, 128), lambda i: (0, 0, 0)),
            pl.BlockSpec((T, C, blk), lambda i: (0, 0, i)),
        ],
        out_specs=pl.BlockSpec((T, C, blk), lambda i: (0, 0, i)),
        compiler_params=pltpu.CompilerParams(
            dimension_semantics=("parallel",)),
    )(mt, b_bc, x_t)

    return jnp.transpose(out_t, (2, 0, 1))


# speedup vs baseline: 156.7980x; 1.1305x over previous
import jax
import jax.numpy as jnp
from jax.experimental import pallas as pl
from jax.experimental.pallas import tpu as pltpu

_HIGHEST = jax.lax.Precision.HIGHEST
_BLOCK_B = 16384


def _wl_body(mt_ref, b_ref, x_ref, o_ref):
    T = x_ref.shape[0]
    nb = x_ref.shape[2]
    mt = mt_ref[...]
    for t in range(T):
        y = jnp.dot(mt, x_ref[t], preferred_element_type=jnp.float32)
        bt = jnp.broadcast_to(b_ref[t][:, 0:1], y.shape)
        o_ref[t] = jnp.maximum(y + bt, 0.0).astype(o_ref.dtype)


def kernel(x, corr_adj, w, bias):
    B, T, C = x.shape
    dtype = x.dtype

    m = jnp.matmul(corr_adj.astype(jnp.float32), w.astype(jnp.float32),
                   precision=_HIGHEST)
    mt = m.T

    bias_f32 = (jnp.zeros((T, C), jnp.float32) if bias is None
                else bias.astype(jnp.float32))
    b_bc = jnp.broadcast_to(bias_f32[:, :, None], (T, C, 128))

    x_t = jnp.transpose(x, (1, 2, 0))
    blk = min(_BLOCK_B, B)
    grid = (pl.cdiv(B, blk),)

    out_t = pl.pallas_call(
        _wl_body,
        out_shape=jax.ShapeDtypeStruct((T, C, B), dtype),
        grid=grid,
        in_specs=[
            pl.BlockSpec((C, C), lambda i: (0, 0)),
            pl.BlockSpec((T, C, 128), lambda i: (0, 0, 0)),
            pl.BlockSpec((T, C, blk), lambda i: (0, 0, i)),
        ],
        out_specs=pl.BlockSpec((T, C, blk), lambda i: (0, 0, i)),
        compiler_params=pltpu.CompilerParams(
            dimension_semantics=("parallel",)),
    )(mt, b_bc, x_t)

    return jnp.transpose(out_t, (2, 0, 1))


# speedup vs baseline: 158.0739x; 1.0081x over previous
import jax
import jax.numpy as jnp
from jax.experimental import pallas as pl
from jax.experimental.pallas import tpu as pltpu

_HIGHEST = jax.lax.Precision.HIGHEST
_BLOCK_B = 32768


def _wl_body(mt_ref, b_ref, x_ref, o_ref):
    T = x_ref.shape[0]
    nb = x_ref.shape[2]
    mt = mt_ref[...]
    for t in range(T):
        y = jnp.dot(mt, x_ref[t], preferred_element_type=jnp.float32)
        bt = jnp.broadcast_to(b_ref[t][:, 0:1], y.shape)
        o_ref[t] = jnp.maximum(y + bt, 0.0).astype(o_ref.dtype)


def kernel(x, corr_adj, w, bias):
    B, T, C = x.shape
    dtype = x.dtype

    m = jnp.matmul(corr_adj.astype(jnp.float32), w.astype(jnp.float32),
                   precision=_HIGHEST)
    mt = m.T

    bias_f32 = (jnp.zeros((T, C), jnp.float32) if bias is None
                else bias.astype(jnp.float32))
    b_bc = jnp.broadcast_to(bias_f32[:, :, None], (T, C, 128))

    x_t = jnp.transpose(x, (1, 2, 0))
    blk = min(_BLOCK_B, B)
    grid = (pl.cdiv(B, blk),)

    out_t = pl.pallas_call(
        _wl_body,
        out_shape=jax.ShapeDtypeStruct((T, C, B), dtype),
        grid=grid,
        in_specs=[
            pl.BlockSpec((C, C), lambda i: (0, 0)),
            pl.BlockSpec((T, C, 128), lambda i: (0, 0, 0)),
            pl.BlockSpec((T, C, blk), lambda i: (0, 0, i)),
        ],
        out_specs=pl.BlockSpec((T, C, blk), lambda i: (0, 0, i)),
        compiler_params=pltpu.CompilerParams(
            dimension_semantics=("parallel",)),
    )(mt, b_bc, x_t)

    return jnp.transpose(out_t, (2, 0, 1))


# speedup vs baseline: 174.4682x; 1.1037x over previous
import jax
import jax.numpy as jnp
from jax import lax
from jax.experimental import pallas as pl
from jax.experimental.pallas import tpu as pltpu

_HIGHEST = jax.lax.Precision.HIGHEST
_BLOCK_B = 32768


def _wl_body(a_ref, w_ref, b_ref, x_ref, o_ref):
    T = x_ref.shape[0]
    m = jnp.dot(a_ref[...], w_ref[...],
                preferred_element_type=jnp.float32, precision=_HIGHEST)
    for t in range(T):
        y = lax.dot_general(m, x_ref[t], (((0,), (0,)), ((), ())),
                            preferred_element_type=jnp.float32)
        bt = jnp.broadcast_to(b_ref[:, t:t + 1], y.shape)
        o_ref[t] = jnp.maximum(y + bt, 0.0).astype(o_ref.dtype)


def kernel(x, corr_adj, w, bias):
    B, T, C = x.shape
    dtype = x.dtype

    bias_ct = (jnp.zeros((C, T), dtype) if bias is None
               else jnp.transpose(bias, (1, 0)))
    x_t = jnp.transpose(x, (1, 2, 0))
    blk = min(_BLOCK_B, B)
    grid = (pl.cdiv(B, blk),)

    out_t = pl.pallas_call(
        _wl_body,
        out_shape=jax.ShapeDtypeStruct((T, C, B), dtype),
        grid=grid,
        in_specs=[
            pl.BlockSpec((C, C), lambda i: (0, 0)),
            pl.BlockSpec((C, C), lambda i: (0, 0)),
            pl.BlockSpec((C, T), lambda i: (0, 0)),
            pl.BlockSpec((T, C, blk), lambda i: (0, 0, i)),
        ],
        out_specs=pl.BlockSpec((T, C, blk), lambda i: (0, 0, i)),
        compiler_params=pltpu.CompilerParams(
            dimension_semantics=("parallel",)),
    )(corr_adj, w, bias_ct, x_t)

    return jnp.transpose(out_t, (2, 0, 1))
